# baseline (speedup 1.0000x reference)
import functools

import jax
import jax.numpy as jnp
from jax.experimental import pallas as pl
from jax.experimental.pallas import tpu as pltpu

_EPS = 1e-6
_R = 512
_WF = 64
_PREC = jax.lax.Precision.HIGHEST


def _onehot(rel, wwin):
    col = jax.lax.broadcasted_iota(jnp.int32, (_R, wwin), 1)
    return (rel == col).astype(jnp.float32)


def _scatter(rel, x, base_al, mom_ref, wwin):
    onehot = _onehot(rel, wwin)
    m = jnp.concatenate([x, x * x, jnp.ones_like(x)], axis=1)
    dn = (((0,), (0,)), ((), ()))
    s = jax.lax.dot_general(onehot, m, dn,
                            preferred_element_type=jnp.float32,
                            precision=_PREC)
    start = pl.multiple_of(base_al, 8)
    mom_ref[pl.ds(start, wwin), :] += s


def _stats_body(base_al_ref, span_ref, rank_ref, x_ref, mom_ref,
                *, nblocks):
    b = pl.program_id(0)

    @pl.when(b == 0)
    def _init():
        mom_ref[...] = jnp.zeros_like(mom_ref)

    @pl.when(b < nblocks)
    def _accum():
        rel = rank_ref[0] - base_al_ref[b]
        x = x_ref[...]

        @pl.when(span_ref[b] <= _WF)
        def _fast():
            _scatter(rel, x, base_al_ref[b], mom_ref, _WF)

        @pl.when(span_ref[b] > _WF)
        def _slow():
            _scatter(rel, x, base_al_ref[b], mom_ref, _R + 8)


def _finalize_body(mom_ref, w_ref, b_ref, fin_ref):
    d = w_ref.shape[1]
    ssum = mom_ref[:, :d]
    ssq = mom_ref[:, d:2 * d]
    cnt = mom_ref[:, 2 * d:2 * d + 1]
    safe = jnp.maximum(cnt, 1.0)
    mean = ssum / safe
    ssd = jnp.maximum(ssq - mean * ssum, 0.0) + _EPS
    var = ssd / (cnt - 1.0)
    std = jnp.sqrt(jnp.maximum(var, 1e-7))
    scale = w_ref[...] / (std + _EPS)
    fin_ref[...] = jnp.concatenate([scale, b_ref[...] - mean * scale], axis=1)


def _norm_body(base_al_ref, span_ref, rank_ref, x_ref, fin_hbm, out_ref,
               fin_vmem, sem, *, nblocks):
    b = pl.program_id(0)

    @pl.when(b == 0)
    def _init():
        pltpu.make_async_copy(fin_hbm, fin_vmem, sem).start()
        pltpu.make_async_copy(fin_hbm, fin_vmem, sem).wait()

    rel = rank_ref[0] - base_al_ref[b]

    def _expand(wwin):
        onehot = _onehot(rel, wwin)
        start = pl.multiple_of(base_al_ref[b], 8)
        window = fin_vmem[pl.ds(start, wwin), :]
        dn = (((1,), (0,)), ((), ()))
        g = jax.lax.dot_general(onehot, window, dn,
                                preferred_element_type=jnp.float32,
                                precision=_PREC)
        d = x_ref.shape[1]
        out_ref[...] = x_ref[...] * g[:, :d] + g[:, d:]

    @pl.when(span_ref[b] <= _WF)
    def _fast():
        _expand(_WF)

    @pl.when(span_ref[b] > _WF)
    def _slow():
        _expand(_R + 8)


def _crystal_norm(target_fea, index, weight, bias, num_segments,
                  interpret=False):
    n, d = target_fea.shape
    nblocks = n // _R
    wslow = _R + 8
    s_pad = ((num_segments + wslow + 7) // 8) * 8

    boundary = jnp.concatenate([
        jnp.zeros((1,), jnp.int32),
        (index[1:] != index[:-1]).astype(jnp.int32)])
    rank = jnp.cumsum(boundary, dtype=jnp.int32)
    rank_base = rank[::_R]
    base_al = rank_base - (rank_base % 8)
    span = rank[_R - 1::_R] - base_al + 1

    rank3 = rank.reshape(nblocks, _R, 1)
    w2 = weight.reshape(1, d).astype(jnp.float32)
    b2 = bias.reshape(1, d).astype(jnp.float32)

    stats_spec = pltpu.PrefetchScalarGridSpec(
        num_scalar_prefetch=2,
        grid=(nblocks + 1,),
        in_specs=[
            pl.BlockSpec((1, _R, 1),
                         lambda b, *_: (jnp.minimum(b, nblocks - 1), 0, 0)),
            pl.BlockSpec((_R, d), lambda b, *_: (jnp.minimum(b, nblocks - 1), 0)),
        ],
        out_specs=pl.BlockSpec((s_pad, 2 * d + 128), lambda b, *_: (0, 0)),
    )
    mom = pl.pallas_call(
        functools.partial(_stats_body, nblocks=nblocks),
        grid_spec=stats_spec,
        out_shape=jax.ShapeDtypeStruct((s_pad, 2 * d + 128), jnp.float32),
        interpret=interpret,
    )(base_al, span, rank3, target_fea)

    fin = pl.pallas_call(
        _finalize_body,
        out_shape=jax.ShapeDtypeStruct((s_pad, 2 * d), jnp.float32),
        interpret=interpret,
    )(mom, w2, b2)

    norm_spec = pltpu.PrefetchScalarGridSpec(
        num_scalar_prefetch=2,
        grid=(nblocks,),
        in_specs=[
            pl.BlockSpec((1, _R, 1), lambda b, *_: (b, 0, 0)),
            pl.BlockSpec((_R, d), lambda b, *_: (b, 0)),
            pl.BlockSpec(memory_space=pltpu.MemorySpace.HBM),
        ],
        out_specs=pl.BlockSpec((_R, d), lambda b, *_: (b, 0)),
        scratch_shapes=[
            pltpu.VMEM((s_pad, 2 * d), jnp.float32),
            pltpu.SemaphoreType.DMA,
        ],
    )
    return pl.pallas_call(
        functools.partial(_norm_body, nblocks=nblocks),
        grid_spec=norm_spec,
        out_shape=jax.ShapeDtypeStruct((n, d), jnp.float32),
        interpret=interpret,
    )(base_al, span, rank3, target_fea, fin)


def kernel(target_fea, index, weight, bias):
    return _crystal_norm(target_fea, index, weight, bias, 10000)


# speedup vs baseline: 1.0014x; 1.0014x over previous
import functools

import jax
import jax.numpy as jnp
from jax.experimental import pallas as pl
from jax.experimental.pallas import tpu as pltpu

_EPS = 1e-6
_R = 512
_WF = 64
_PREC = jax.lax.Precision.HIGHEST


def _onehot(rel, wwin):
    col = jax.lax.broadcasted_iota(jnp.int32, (_R, wwin), 1)
    return (rel == col).astype(jnp.float32)


def _scatter(rel, x, base_al, mom_ref, wwin):
    onehot = _onehot(rel, wwin)
    m = jnp.concatenate([x, x * x, jnp.ones_like(x)], axis=1)
    dn = (((0,), (0,)), ((), ()))
    s = jax.lax.dot_general(onehot, m, dn,
                            preferred_element_type=jnp.float32,
                            precision=_PREC)
    start = pl.multiple_of(base_al, 8)
    mom_ref[pl.ds(start, wwin), :] += s


def _stats_body(base_al_ref, span_ref, rank_ref, x_ref, mom_ref,
                *, nblocks):
    b = pl.program_id(0)

    @pl.when(b == 0)
    def _init():
        mom_ref[...] = jnp.zeros_like(mom_ref)

    @pl.when(b < nblocks)
    def _accum():
        rel = rank_ref[0] - base_al_ref[b]
        x = x_ref[...]

        @pl.when(span_ref[b] <= _WF)
        def _fast():
            _scatter(rel, x, base_al_ref[b], mom_ref, _WF)

        @pl.when(span_ref[b] > _WF)
        def _slow():
            _scatter(rel, x, base_al_ref[b], mom_ref, _R + 8)


def _finalize_body(mom_ref, w_ref, b_ref, fin_ref):
    d = w_ref.shape[1]
    ssum = mom_ref[:, :d]
    ssq = mom_ref[:, d:2 * d]
    cnt = mom_ref[:, 2 * d:2 * d + 1]
    safe = jnp.maximum(cnt, 1.0)
    mean = ssum / safe
    ssd = jnp.maximum(ssq - mean * ssum, 0.0) + _EPS
    var = ssd / (cnt - 1.0)
    std = jnp.sqrt(jnp.maximum(var, 1e-7))
    scale = w_ref[...] / (std + _EPS)
    fin_ref[...] = jnp.concatenate([scale, b_ref[...] - mean * scale], axis=1)


def _norm_body(base_al_ref, span_ref, rank_ref, x_ref, fin_vmem, out_ref,
               *, nblocks):
    b = pl.program_id(0)

    rel = rank_ref[0] - base_al_ref[b]

    def _expand(wwin):
        onehot = _onehot(rel, wwin)
        start = pl.multiple_of(base_al_ref[b], 8)
        window = fin_vmem[pl.ds(start, wwin), :]
        dn = (((1,), (0,)), ((), ()))
        g = jax.lax.dot_general(onehot, window, dn,
                                preferred_element_type=jnp.float32,
                                precision=_PREC)
        d = x_ref.shape[1]
        out_ref[...] = x_ref[...] * g[:, :d] + g[:, d:]

    @pl.when(span_ref[b] <= _WF)
    def _fast():
        _expand(_WF)

    @pl.when(span_ref[b] > _WF)
    def _slow():
        _expand(_R + 8)


def _crystal_norm(target_fea, index, weight, bias, num_segments,
                  interpret=False):
    n, d = target_fea.shape
    nblocks = n // _R
    wslow = _R + 8
    s_pad = ((num_segments + wslow + 7) // 8) * 8

    boundary = jnp.concatenate([
        jnp.zeros((1,), jnp.int32),
        (index[1:] != index[:-1]).astype(jnp.int32)])
    rank = jnp.cumsum(boundary, dtype=jnp.int32)
    rank_base = rank[::_R]
    base_al = rank_base - (rank_base % 8)
    span = rank[_R - 1::_R] - base_al + 1

    rank3 = rank.reshape(nblocks, _R, 1)
    w2 = weight.reshape(1, d).astype(jnp.float32)
    b2 = bias.reshape(1, d).astype(jnp.float32)

    stats_spec = pltpu.PrefetchScalarGridSpec(
        num_scalar_prefetch=2,
        grid=(nblocks + 1,),
        in_specs=[
            pl.BlockSpec((1, _R, 1),
                         lambda b, *_: (jnp.minimum(b, nblocks - 1), 0, 0)),
            pl.BlockSpec((_R, d), lambda b, *_: (jnp.minimum(b, nblocks - 1), 0)),
        ],
        out_specs=pl.BlockSpec((s_pad, 2 * d + 128), lambda b, *_: (0, 0)),
    )
    mom = pl.pallas_call(
        functools.partial(_stats_body, nblocks=nblocks),
        grid_spec=stats_spec,
        out_shape=jax.ShapeDtypeStruct((s_pad, 2 * d + 128), jnp.float32),
        interpret=interpret,
    )(base_al, span, rank3, target_fea)

    fin = pl.pallas_call(
        _finalize_body,
        out_shape=jax.ShapeDtypeStruct((s_pad, 2 * d), jnp.float32),
        interpret=interpret,
    )(mom, w2, b2)

    norm_spec = pltpu.PrefetchScalarGridSpec(
        num_scalar_prefetch=2,
        grid=(nblocks,),
        in_specs=[
            pl.BlockSpec((1, _R, 1), lambda b, *_: (b, 0, 0)),
            pl.BlockSpec((_R, d), lambda b, *_: (b, 0)),
            pl.BlockSpec((s_pad, 2 * d), lambda b, *_: (0, 0)),
        ],
        out_specs=pl.BlockSpec((_R, d), lambda b, *_: (b, 0)),
    )
    return pl.pallas_call(
        functools.partial(_norm_body, nblocks=nblocks),
        grid_spec=norm_spec,
        out_shape=jax.ShapeDtypeStruct((n, d), jnp.float32),
        interpret=interpret,
    )(base_al, span, rank3, target_fea, fin)


def kernel(target_fea, index, weight, bias):
    return _crystal_norm(target_fea, index, weight, bias, 10000)


# speedup vs baseline: 1.0824x; 1.0809x over previous
import functools

import jax
import jax.numpy as jnp
from jax.experimental import pallas as pl
from jax.experimental.pallas import tpu as pltpu

_EPS = 1e-6
_R = 800
_WF = 64
_PREC = jax.lax.Precision.HIGHEST


def _onehot(rel, wwin):
    col = jax.lax.broadcasted_iota(jnp.int32, (_R, wwin), 1)
    return (rel == col).astype(jnp.float32)


def _scatter(rel, x, base_al, mom_ref, wwin):
    onehot = _onehot(rel, wwin)
    m = jnp.concatenate([x, x * x, jnp.ones_like(x)], axis=1)
    dn = (((0,), (0,)), ((), ()))
    s = jax.lax.dot_general(onehot, m, dn,
                            preferred_element_type=jnp.float32,
                            precision=_PREC)
    start = pl.multiple_of(base_al, 8)
    mom_ref[pl.ds(start, wwin), :] += s


def _stats_body(base_al_ref, span_ref, rank_ref, x_ref, mom_ref,
                *, nblocks):
    b = pl.program_id(0)

    @pl.when(b == 0)
    def _init():
        mom_ref[...] = jnp.zeros_like(mom_ref)

    @pl.when(b < nblocks)
    def _accum():
        rel = rank_ref[0] - base_al_ref[b]
        x = x_ref[...]

        @pl.when(span_ref[b] <= _WF)
        def _fast():
            _scatter(rel, x, base_al_ref[b], mom_ref, _WF)

        @pl.when(span_ref[b] > _WF)
        def _slow():
            _scatter(rel, x, base_al_ref[b], mom_ref, _R + 8)


def _finalize_body(mom_ref, w_ref, b_ref, fin_ref):
    d = w_ref.shape[1]
    ssum = mom_ref[:, :d]
    ssq = mom_ref[:, d:2 * d]
    cnt = mom_ref[:, 2 * d:2 * d + 1]
    safe = jnp.maximum(cnt, 1.0)
    mean = ssum / safe
    ssd = jnp.maximum(ssq - mean * ssum, 0.0) + _EPS
    var = ssd / (cnt - 1.0)
    std = jnp.sqrt(jnp.maximum(var, 1e-7))
    scale = w_ref[...] / (std + _EPS)
    fin_ref[...] = jnp.concatenate([scale, b_ref[...] - mean * scale], axis=1)


def _norm_body(base_al_ref, span_ref, rank_ref, x_ref, fin_vmem, out_ref,
               *, nblocks):
    b = pl.program_id(0)

    rel = rank_ref[0] - base_al_ref[b]

    def _expand(wwin):
        onehot = _onehot(rel, wwin)
        start = pl.multiple_of(base_al_ref[b], 8)
        window = fin_vmem[pl.ds(start, wwin), :]
        dn = (((1,), (0,)), ((), ()))
        g = jax.lax.dot_general(onehot, window, dn,
                                preferred_element_type=jnp.float32,
                                precision=_PREC)
        d = x_ref.shape[1]
        out_ref[...] = x_ref[...] * g[:, :d] + g[:, d:]

    @pl.when(span_ref[b] <= _WF)
    def _fast():
        _expand(_WF)

    @pl.when(span_ref[b] > _WF)
    def _slow():
        _expand(_R + 8)


def _crystal_norm(target_fea, index, weight, bias, num_segments,
                  interpret=False):
    n, d = target_fea.shape
    nblocks = n // _R
    wslow = _R + 8
    s_pad = ((num_segments + wslow + 7) // 8) * 8

    boundary = jnp.concatenate([
        jnp.zeros((1,), jnp.int32),
        (index[1:] != index[:-1]).astype(jnp.int32)])
    rank = jnp.cumsum(boundary, dtype=jnp.int32)
    rank_base = rank[::_R]
    base_al = rank_base - (rank_base % 8)
    span = rank[_R - 1::_R] - base_al + 1

    rank3 = rank.reshape(nblocks, _R, 1)
    w2 = weight.reshape(1, d).astype(jnp.float32)
    b2 = bias.reshape(1, d).astype(jnp.float32)

    stats_spec = pltpu.PrefetchScalarGridSpec(
        num_scalar_prefetch=2,
        grid=(nblocks + 1,),
        in_specs=[
            pl.BlockSpec((1, _R, 1),
                         lambda b, *_: (jnp.minimum(b, nblocks - 1), 0, 0)),
            pl.BlockSpec((_R, d), lambda b, *_: (jnp.minimum(b, nblocks - 1), 0)),
        ],
        out_specs=pl.BlockSpec((s_pad, 2 * d + 128), lambda b, *_: (0, 0)),
    )
    mom = pl.pallas_call(
        functools.partial(_stats_body, nblocks=nblocks),
        grid_spec=stats_spec,
        out_shape=jax.ShapeDtypeStruct((s_pad, 2 * d + 128), jnp.float32),
        interpret=interpret,
    )(base_al, span, rank3, target_fea)

    fin = pl.pallas_call(
        _finalize_body,
        out_shape=jax.ShapeDtypeStruct((s_pad, 2 * d), jnp.float32),
        interpret=interpret,
    )(mom, w2, b2)

    norm_spec = pltpu.PrefetchScalarGridSpec(
        num_scalar_prefetch=2,
        grid=(nblocks,),
        in_specs=[
            pl.BlockSpec((1, _R, 1), lambda b, *_: (b, 0, 0)),
            pl.BlockSpec((_R, d), lambda b, *_: (b, 0)),
            pl.BlockSpec((s_pad, 2 * d), lambda b, *_: (0, 0)),
        ],
        out_specs=pl.BlockSpec((_R, d), lambda b, *_: (b, 0)),
    )
    return pl.pallas_call(
        functools.partial(_norm_body, nblocks=nblocks),
        grid_spec=norm_spec,
        out_shape=jax.ShapeDtypeStruct((n, d), jnp.float32),
        interpret=interpret,
    )(base_al, span, rank3, target_fea, fin)


def kernel(target_fea, index, weight, bias):
    return _crystal_norm(target_fea, index, weight, bias, 10000)


# speedup vs baseline: 1.1201x; 1.0349x over previous
import functools

import jax
import jax.numpy as jnp
from jax.experimental import pallas as pl
from jax.experimental.pallas import tpu as pltpu

_EPS = 1e-6
_R = 800
_WF = 64
_PREC = jax.lax.Precision.HIGHEST


def _onehot(rel, wwin):
    col = jax.lax.broadcasted_iota(jnp.int32, (_R, wwin), 1)
    return (rel == col).astype(jnp.float32)


def _scatter(rel, x, base_al, mom_ref, wwin):
    onehot = _onehot(rel, wwin)
    m = jnp.concatenate([x, x * x], axis=1)
    dn = (((0,), (0,)), ((), ()))
    s = jax.lax.dot_general(onehot, m, dn,
                            preferred_element_type=jnp.float32,
                            precision=_PREC)
    c = jax.lax.dot_general(onehot, jnp.ones_like(x), dn,
                            preferred_element_type=jnp.float32,
                            precision=jax.lax.Precision.DEFAULT)
    start = pl.multiple_of(base_al, 8)
    mom_ref[pl.ds(start, wwin), :] += jnp.concatenate([s, c], axis=1)


def _stats_body(base_al_ref, span_ref, rank_ref, x_ref, mom_ref,
                *, nblocks):
    b = pl.program_id(0)

    @pl.when(b == 0)
    def _init():
        mom_ref[...] = jnp.zeros_like(mom_ref)

    @pl.when(b < nblocks)
    def _accum():
        rel = rank_ref[0] - base_al_ref[b]
        x = x_ref[...]

        @pl.when(span_ref[b] <= _WF)
        def _fast():
            _scatter(rel, x, base_al_ref[b], mom_ref, _WF)

        @pl.when(span_ref[b] > _WF)
        def _slow():
            _scatter(rel, x, base_al_ref[b], mom_ref, _R + 8)


def _finalize_body(mom_ref, w_ref, b_ref, fin_ref):
    d = w_ref.shape[1]
    ssum = mom_ref[:, :d]
    ssq = mom_ref[:, d:2 * d]
    cnt = mom_ref[:, 2 * d:2 * d + 1]
    safe = jnp.maximum(cnt, 1.0)
    mean = ssum / safe
    ssd = jnp.maximum(ssq - mean * ssum, 0.0) + _EPS
    var = ssd / (cnt - 1.0)
    std = jnp.sqrt(jnp.maximum(var, 1e-7))
    scale = w_ref[...] / (std + _EPS)
    fin_ref[...] = jnp.concatenate([scale, b_ref[...] - mean * scale], axis=1)


def _norm_body(base_al_ref, span_ref, rank_ref, x_ref, fin_vmem, out_ref,
               *, nblocks):
    b = pl.program_id(0)

    rel = rank_ref[0] - base_al_ref[b]

    def _expand(wwin):
        onehot = _onehot(rel, wwin)
        start = pl.multiple_of(base_al_ref[b], 8)
        window = fin_vmem[pl.ds(start, wwin), :]
        dn = (((1,), (0,)), ((), ()))
        g = jax.lax.dot_general(onehot, window, dn,
                                preferred_element_type=jnp.float32,
                                precision=_PREC)
        d = x_ref.shape[1]
        out_ref[...] = x_ref[...] * g[:, :d] + g[:, d:]

    @pl.when(span_ref[b] <= _WF)
    def _fast():
        _expand(_WF)

    @pl.when(span_ref[b] > _WF)
    def _slow():
        _expand(_R + 8)


def _crystal_norm(target_fea, index, weight, bias, num_segments,
                  interpret=False):
    n, d = target_fea.shape
    nblocks = n // _R
    wslow = _R + 8
    s_pad = ((num_segments + wslow + 7) // 8) * 8

    boundary = jnp.concatenate([
        jnp.zeros((1,), jnp.int32),
        (index[1:] != index[:-1]).astype(jnp.int32)])
    rank = jnp.cumsum(boundary, dtype=jnp.int32)
    rank_base = rank[::_R]
    base_al = rank_base - (rank_base % 8)
    span = rank[_R - 1::_R] - base_al + 1

    rank3 = rank.reshape(nblocks, _R, 1)
    w2 = weight.reshape(1, d).astype(jnp.float32)
    b2 = bias.reshape(1, d).astype(jnp.float32)

    stats_spec = pltpu.PrefetchScalarGridSpec(
        num_scalar_prefetch=2,
        grid=(nblocks + 1,),
        in_specs=[
            pl.BlockSpec((1, _R, 1),
                         lambda b, *_: (jnp.minimum(b, nblocks - 1), 0, 0)),
            pl.BlockSpec((_R, d), lambda b, *_: (jnp.minimum(b, nblocks - 1), 0)),
        ],
        out_specs=pl.BlockSpec((s_pad, 2 * d + 128), lambda b, *_: (0, 0)),
    )
    mom = pl.pallas_call(
        functools.partial(_stats_body, nblocks=nblocks),
        grid_spec=stats_spec,
        out_shape=jax.ShapeDtypeStruct((s_pad, 2 * d + 128), jnp.float32),
        interpret=interpret,
    )(base_al, span, rank3, target_fea)

    fin = pl.pallas_call(
        _finalize_body,
        out_shape=jax.ShapeDtypeStruct((s_pad, 2 * d), jnp.float32),
        interpret=interpret,
    )(mom, w2, b2)

    norm_spec = pltpu.PrefetchScalarGridSpec(
        num_scalar_prefetch=2,
        grid=(nblocks,),
        in_specs=[
            pl.BlockSpec((1, _R, 1), lambda b, *_: (b, 0, 0)),
            pl.BlockSpec((_R, d), lambda b, *_: (b, 0)),
            pl.BlockSpec((s_pad, 2 * d), lambda b, *_: (0, 0)),
        ],
        out_specs=pl.BlockSpec((_R, d), lambda b, *_: (b, 0)),
    )
    return pl.pallas_call(
        functools.partial(_norm_body, nblocks=nblocks),
        grid_spec=norm_spec,
        out_shape=jax.ShapeDtypeStruct((n, d), jnp.float32),
        interpret=interpret,
    )(base_al, span, rank3, target_fea, fin)


def kernel(target_fea, index, weight, bias):
    return _crystal_norm(target_fea, index, weight, bias, 10000)


# speedup vs baseline: 1.1218x; 1.0014x over previous
import functools

import jax
import jax.numpy as jnp
from jax.experimental import pallas as pl
from jax.experimental.pallas import tpu as pltpu

_EPS = 1e-6
_R = 800
_WF = 64
_PREC = jax.lax.Precision.HIGHEST


def _onehot(rel, wwin):
    col = jax.lax.broadcasted_iota(jnp.int32, (_R, wwin), 1)
    return (rel == col).astype(jnp.float32)


def _scatter(rel, x, base_al, mom_ref, wwin):
    onehot = _onehot(rel, wwin)
    m = jnp.concatenate([x, x * x], axis=1)
    dn = (((0,), (0,)), ((), ()))
    s = jax.lax.dot_general(onehot, m, dn,
                            preferred_element_type=jnp.float32,
                            precision=_PREC)
    c = jax.lax.dot_general(onehot, jnp.ones_like(x), dn,
                            preferred_element_type=jnp.float32,
                            precision=jax.lax.Precision.DEFAULT)
    start = pl.multiple_of(base_al, 8)
    mom_ref[pl.ds(start, wwin), :] += jnp.concatenate([s, c], axis=1)


def _stats_body(base_al_ref, span_ref, rank_ref, x_ref, mom_ref,
                *, nblocks):
    b = pl.program_id(0)

    @pl.when(b == 0)
    def _init():
        mom_ref[...] = jnp.zeros_like(mom_ref)

    @pl.when(b < nblocks)
    def _accum():
        rel = rank_ref[0] - base_al_ref[b]
        x = x_ref[...]

        @pl.when(span_ref[b] <= _WF)
        def _fast():
            _scatter(rel, x, base_al_ref[b], mom_ref, _WF)

        @pl.when(span_ref[b] > _WF)
        def _slow():
            _scatter(rel, x, base_al_ref[b], mom_ref, _R + 8)


def _finalize_body(mom_ref, w_ref, b_ref, fin_ref):
    d = w_ref.shape[1]
    ssum = mom_ref[:, :d]
    ssq = mom_ref[:, d:2 * d]
    cnt = mom_ref[:, 2 * d:2 * d + 1]
    safe = jnp.maximum(cnt, 1.0)
    mean = ssum / safe
    ssd = jnp.maximum(ssq - mean * ssum, 0.0) + _EPS
    var = ssd / (cnt - 1.0)
    std = jnp.sqrt(jnp.maximum(var, 1e-7))
    scale = w_ref[...] / (std + _EPS)
    fin_ref[...] = jnp.concatenate([scale, b_ref[...] - mean * scale], axis=1)


def _norm_body(base_al_ref, span_ref, rank_ref, x_ref, fin_vmem, out_ref,
               *, nblocks):
    b = pl.program_id(0)

    rel = rank_ref[0] - base_al_ref[b]

    def _expand(wwin):
        onehot = _onehot(rel, wwin)
        start = pl.multiple_of(base_al_ref[b], 8)
        window = fin_vmem[pl.ds(start, wwin), :]
        dn = (((1,), (0,)), ((), ()))
        g = jax.lax.dot_general(onehot, window, dn,
                                preferred_element_type=jnp.float32,
                                precision=_PREC)
        d = x_ref.shape[1]
        out_ref[...] = x_ref[...] * g[:, :d] + g[:, d:]

    @pl.when(span_ref[b] <= _WF)
    def _fast():
        _expand(_WF)

    @pl.when(span_ref[b] > _WF)
    def _slow():
        _expand(_R + 8)


def _crystal_norm(target_fea, index, weight, bias, num_segments,
                  interpret=False):
    n, d = target_fea.shape
    nblocks = n // _R
    wslow = _R + 8
    s_pad = ((num_segments + wslow + 7) // 8) * 8

    boundary = jnp.concatenate([
        jnp.zeros((1,), jnp.int32),
        (index[1:] != index[:-1]).astype(jnp.int32)])
    rank = jnp.cumsum(boundary, dtype=jnp.int32)
    rank_base = rank[::_R]
    base_al = rank_base - (rank_base % 8)
    span = rank[_R - 1::_R] - base_al + 1

    rank3 = rank.reshape(nblocks, _R, 1)
    w2 = weight.reshape(1, d).astype(jnp.float32)
    b2 = bias.reshape(1, d).astype(jnp.float32)

    stats_spec = pltpu.PrefetchScalarGridSpec(
        num_scalar_prefetch=2,
        grid=(nblocks + 1,),
        in_specs=[
            pl.BlockSpec((1, _R, 1),
                         lambda b, *_: (jnp.minimum(b, nblocks - 1), 0, 0)),
            pl.BlockSpec((_R, d), lambda b, *_: (jnp.minimum(b, nblocks - 1), 0)),
        ],
        out_specs=pl.BlockSpec((s_pad, 2 * d + 128), lambda b, *_: (0, 0)),
    )
    mom = pl.pallas_call(
        functools.partial(_stats_body, nblocks=nblocks),
        grid_spec=stats_spec,
        out_shape=jax.ShapeDtypeStruct((s_pad, 2 * d + 128), jnp.float32),
        interpret=interpret,
    )(base_al, span, rank3, target_fea)

    fin = pl.pallas_call(
        _finalize_body,
        out_shape=jax.ShapeDtypeStruct((s_pad, 2 * d), jnp.float32),
        interpret=interpret,
    )(mom, w2, b2)

    norm_spec = pltpu.PrefetchScalarGridSpec(
        num_scalar_prefetch=2,
        grid=(nblocks,),
        in_specs=[
            pl.BlockSpec((1, _R, 1), lambda b, *_: (b, 0, 0)),
            pl.BlockSpec((_R, d), lambda b, *_: (b, 0)),
            pl.BlockSpec((s_pad, 2 * d), lambda b, *_: (0, 0)),
        ],
        out_specs=pl.BlockSpec((_R, d), lambda b, *_: (b, 0)),
    )
    return pl.pallas_call(
        functools.partial(_norm_body, nblocks=nblocks),
        grid_spec=norm_spec,
        out_shape=jax.ShapeDtypeStruct((n, d), jnp.float32),
        compiler_params=pltpu.CompilerParams(
            dimension_semantics=("parallel",)),
        interpret=interpret,
    )(base_al, span, rank3, target_fea, fin)


def kernel(target_fea, index, weight, bias):
    return _crystal_norm(target_fea, index, weight, bias, 10000)
